# Initial kernel scaffold; baseline (speedup 1.0000x reference)
import jax
import jax.numpy as jnp
from jax.experimental import pallas as pl


def kernel(x, W):
    raise NotImplementedError("write your pallas kernel here")


# speedup vs baseline: 1.2791x; 1.2791x over previous
import functools

import jax
import jax.numpy as jnp
from jax import lax
from jax.experimental import pallas as pl
from jax.experimental.pallas import tpu as pltpu

K_NB = 128
LANES = 128
_BIG_IDX = 2**30


def _lex_less(av, ai, bv, bi):
    return (av < bv) | ((av == bv) & (ai < bi))


def _partner(v, j):
    rp = jnp.roll(v, -j, axis=-1)
    rm = jnp.roll(v, j, axis=-1)
    return rp, rm


def _cmpex(v, i, lane, j, up):
    vp_p, vp_m = _partner(v, j)
    ip_p, ip_m = _partner(i, j)
    lower = (lane & j) == 0
    pv = jnp.where(lower, vp_p, vp_m)
    pi = jnp.where(lower, ip_p, ip_m)
    less = _lex_less(v, i, pv, pi)
    keep_self = (lower == up) == less
    return jnp.where(keep_self, v, pv), jnp.where(keep_self, i, pi)


def _sort_lanes(v, i, lane, ascending):
    for kk in (2, 4, 8, 16, 32, 64, 128):
        blk_up = (lane & kk) == 0
        if not ascending:
            blk_up = jnp.logical_not(blk_up)
        jj = kk // 2
        while jj >= 1:
            v, i = _cmpex(v, i, lane, jj, blk_up)
            jj //= 2
    return v, i


def _merge_desc_into_asc(bufv, bufi, cv, ci, lane):
    less = _lex_less(bufv, bufi, cv, ci)
    nv = jnp.where(less, bufv, cv)
    ni = jnp.where(less, bufi, ci)
    true_up = lane >= 0
    for j in (64, 32, 16, 8, 4, 2, 1):
        nv, ni = _cmpex(nv, ni, lane, j, true_up)
    return nv, ni


def _topk_body(n_chunks, v_real, x_ref, w_ref, xsq_ref, wsq_ref,
               d_ref, i_ref, bufv, bufi):
    j = pl.program_id(1)

    @pl.when(j == 0)
    def _init():
        bufv[...] = jnp.full(bufv.shape, jnp.inf, jnp.float32)
        bufi[...] = jnp.full(bufi.shape, _BIG_IDX, jnp.int32)

    xb = x_ref[...]
    wb = w_ref[...]
    wsq = wsq_ref[0]
    xsq = xsq_ref[...]
    mm = lax.dot_general(
        xb, wb, (((1,), (1,)), ((), ())),
        preferred_element_type=jnp.float32)
    s = (xsq - 2.0 * mm) + wsq

    shape = s.shape
    lane = lax.broadcasted_iota(jnp.int32, shape, 1)
    col = lane + j * LANES
    s = jnp.where(col < v_real, s, jnp.inf)

    cv, ci = _sort_lanes(s, col, lane, ascending=False)
    nv, ni = _merge_desc_into_asc(bufv[...], bufi[...], cv, ci, lane)
    bufv[...] = nv
    bufi[...] = ni

    @pl.when(j == n_chunks - 1)
    def _fin():
        d_ref[...] = bufv[...]
        i_ref[...] = bufi[...]


def kernel(x, W):
    B, d = x.shape
    V, _ = W.shape
    n_chunks = -(-V // LANES)
    v_pad = n_chunks * LANES
    xsq = jnp.sum(x * x, axis=1, keepdims=True)
    wsq = jnp.sum(W * W, axis=1)
    if v_pad != V:
        W = jnp.pad(W, ((0, v_pad - V), (0, 0)))
        wsq = jnp.pad(wsq, (0, v_pad - V))
    wsq3 = wsq.reshape(n_chunks, 1, LANES)
    bb = 256
    grid = (B // bb, n_chunks)

    body = functools.partial(_topk_body, n_chunks, V)
    D, I = pl.pallas_call(
        body,
        grid=grid,
        in_specs=[
            pl.BlockSpec((bb, d), lambda i, j: (i, 0)),
            pl.BlockSpec((LANES, d), lambda i, j: (j, 0)),
            pl.BlockSpec((bb, 1), lambda i, j: (i, 0)),
            pl.BlockSpec((1, 1, LANES), lambda i, j: (j, 0, 0)),
        ],
        out_specs=[
            pl.BlockSpec((bb, K_NB), lambda i, j: (i, 0)),
            pl.BlockSpec((bb, K_NB), lambda i, j: (i, 0)),
        ],
        out_shape=[
            jax.ShapeDtypeStruct((B, K_NB), jnp.float32),
            jax.ShapeDtypeStruct((B, K_NB), jnp.int32),
        ],
        scratch_shapes=[
            pltpu.VMEM((bb, LANES), jnp.float32),
            pltpu.VMEM((bb, LANES), jnp.int32),
        ],
        compiler_params=pltpu.CompilerParams(
            dimension_semantics=("arbitrary", "arbitrary")),
    )(x, W, xsq, wsq3)
    return x, D, I


# speedup vs baseline: 3.8898x; 3.0410x over previous
import functools

import jax
import jax.numpy as jnp
from jax import lax
from jax.experimental import pallas as pl
from jax.experimental.pallas import tpu as pltpu
from jax.experimental.pallas import tpu_sc as plsc

K_NB = 128
LANES = 128
_BIG_IDX = 2**30
SS = 8
CAP = 2048
SC_NC = 2
SC_NS = 16
SC_L = 16


def _lex_less(av, ai, bv, bi):
    return (av < bv) | ((av == bv) & (ai < bi))


def _partner(v, j):
    rp = jnp.roll(v, -j, axis=-1)
    rm = jnp.roll(v, j, axis=-1)
    return rp, rm


def _cmpex(v, i, lane, j, up):
    vp_p, vp_m = _partner(v, j)
    ip_p, ip_m = _partner(i, j)
    lower = (lane & j) == 0
    pv = jnp.where(lower, vp_p, vp_m)
    pi = jnp.where(lower, ip_p, ip_m)
    less = _lex_less(v, i, pv, pi)
    keep_self = (lower == up) == less
    return jnp.where(keep_self, v, pv), jnp.where(keep_self, i, pi)


def _sort_lanes(v, i, lane, ascending):
    for kk in (2, 4, 8, 16, 32, 64, 128):
        blk_up = (lane & kk) == 0
        if not ascending:
            blk_up = jnp.logical_not(blk_up)
        jj = kk // 2
        while jj >= 1:
            v, i = _cmpex(v, i, lane, jj, blk_up)
            jj //= 2
    return v, i


def _merge_desc_into_asc(bufv, bufi, cv, ci, lane):
    less = _lex_less(bufv, bufi, cv, ci)
    nv = jnp.where(less, bufv, cv)
    ni = jnp.where(less, bufi, ci)
    true_up = lane >= 0
    for j in (64, 32, 16, 8, 4, 2, 1):
        nv, ni = _cmpex(nv, ni, lane, j, true_up)
    return nv, ni


def _score_body(n_chunks, v_real, x_ref, w_ref, xsq_ref, wsq_ref,
                d2_ref, t_ref, bufv, bufi):
    j = pl.program_id(1)

    @pl.when(j == 0)
    def _init():
        bufv[...] = jnp.full(bufv.shape, jnp.inf, jnp.float32)
        bufi[...] = jnp.full(bufi.shape, _BIG_IDX, jnp.int32)

    xb = x_ref[...]
    wb = w_ref[...]
    wsq = wsq_ref[0]
    xsq = xsq_ref[...]
    mm = lax.dot_general(
        xb, wb, (((1,), (1,)), ((), ())),
        preferred_element_type=jnp.float32)
    s = (xsq - 2.0 * mm) + wsq

    shape = s.shape
    lane = lax.broadcasted_iota(jnp.int32, shape, 1)
    col = lane + j * LANES
    s = jnp.where(col < v_real, s, jnp.inf)
    d2_ref[...] = s

    @pl.when(j % SS == 0)
    def _merge():
        cv, ci = _sort_lanes(s, col, lane, ascending=False)
        nv, ni = _merge_desc_into_asc(bufv[...], bufi[...], cv, ci, lane)
        bufv[...] = nv
        bufi[...] = ni

    @pl.when(j == n_chunks - 1)
    def _fin():
        t_ref[...] = bufv[:, K_NB - 1:K_NB]


def _sc_filter_body(qpw, vpad, d2_hbm, t_hbm, cv_hbm, ci_hbm,
                    row_v, t_v, cvb, cib):
    wid = lax.axis_index("s") * SC_NC + lax.axis_index("c")
    base = wid * qpw
    pltpu.sync_copy(t_hbm.at[pl.ds(base, qpw)], t_v)

    inf16 = jnp.full((SC_L,), jnp.inf, jnp.float32)
    big16 = jnp.full((SC_L,), _BIG_IDX, jnp.int32)
    iota16 = lax.iota(jnp.int32, SC_L)

    def per_query(qi, carry):
        qg = base + qi
        pltpu.sync_copy(d2_hbm.at[qg], row_v)
        tblk = t_v[pl.ds((qi // SC_L) * SC_L, SC_L)]
        tsel = jnp.where(iota16 == qi % SC_L, tblk, -jnp.inf)
        thr = jnp.max(tsel)

        def init_step(n, c):
            cvb[pl.ds(n * SC_L, SC_L)] = inf16
            cib[pl.ds(n * SC_L, SC_L)] = big16
            return c
        lax.fori_loop(0, (CAP + SC_L) // SC_L, init_step, 0)

        def scan_step(n, ptr):
            v = row_v[pl.ds(n * SC_L, SC_L)]
            m = v <= thr
            gi = iota16 + n * SC_L
            plsc.store_compressed(cvb.at[pl.ds(ptr, SC_L)], v, mask=m)
            plsc.store_compressed(cib.at[pl.ds(ptr, SC_L)], gi, mask=m)
            cnt = jnp.max(plsc.all_reduce_population_count(m))
            return jnp.minimum(ptr + cnt, CAP)
        lax.fori_loop(0, vpad // SC_L, scan_step, jnp.int32(0))

        pltpu.sync_copy(cvb.at[pl.ds(0, CAP)], cv_hbm.at[qg])
        pltpu.sync_copy(cib.at[pl.ds(0, CAP)], ci_hbm.at[qg])
        return carry

    lax.fori_loop(0, qpw, per_query, 0)


def _finish_body(n_chunks, cv_ref, ci_ref, d_ref, i_ref, bufv, bufi):
    j = pl.program_id(1)

    @pl.when(j == 0)
    def _init():
        bufv[...] = jnp.full(bufv.shape, jnp.inf, jnp.float32)
        bufi[...] = jnp.full(bufi.shape, _BIG_IDX, jnp.int32)

    v = cv_ref[...]
    idx = ci_ref[...]
    lane = lax.broadcasted_iota(jnp.int32, v.shape, 1)
    cv, ci = _sort_lanes(v, idx, lane, ascending=False)
    nv, ni = _merge_desc_into_asc(bufv[...], bufi[...], cv, ci, lane)
    bufv[...] = nv
    bufi[...] = ni

    @pl.when(j == n_chunks - 1)
    def _fin():
        d_ref[...] = bufv[...]
        i_ref[...] = bufi[...]


def kernel(x, W):
    B, d = x.shape
    V, _ = W.shape
    n_chunks = -(-V // LANES)
    if n_chunks % SS:
        n_chunks += SS - n_chunks % SS
    v_pad = n_chunks * LANES
    xsq = jnp.sum(x * x, axis=1, keepdims=True)
    wsq = jnp.sum(W * W, axis=1)
    if v_pad != V:
        W = jnp.pad(W, ((0, v_pad - V), (0, 0)))
        wsq = jnp.pad(wsq, (0, v_pad - V))
    wsq3 = wsq.reshape(n_chunks, 1, LANES)
    bb = 256
    grid = (B // bb, n_chunks)

    body = functools.partial(_score_body, n_chunks, V)
    d2, t = pl.pallas_call(
        body,
        grid=grid,
        in_specs=[
            pl.BlockSpec((bb, d), lambda i, j: (i, 0)),
            pl.BlockSpec((LANES, d), lambda i, j: (j, 0)),
            pl.BlockSpec((bb, 1), lambda i, j: (i, 0)),
            pl.BlockSpec((1, 1, LANES), lambda i, j: (j, 0, 0)),
        ],
        out_specs=[
            pl.BlockSpec((bb, LANES), lambda i, j: (i, j)),
            pl.BlockSpec((bb, 1), lambda i, j: (i, 0)),
        ],
        out_shape=[
            jax.ShapeDtypeStruct((B, v_pad), jnp.float32),
            jax.ShapeDtypeStruct((B, 1), jnp.float32),
        ],
        scratch_shapes=[
            pltpu.VMEM((bb, LANES), jnp.float32),
            pltpu.VMEM((bb, LANES), jnp.int32),
        ],
        compiler_params=pltpu.CompilerParams(
            dimension_semantics=("arbitrary", "arbitrary")),
    )(x, W, xsq, wsq3)

    n_workers = SC_NC * SC_NS
    qpw = B // n_workers
    mesh = plsc.VectorSubcoreMesh(core_axis_name="c", subcore_axis_name="s")
    sc_body = functools.partial(_sc_filter_body, qpw, v_pad)
    cand_v, cand_i = pl.kernel(
        sc_body,
        out_type=[
            jax.ShapeDtypeStruct((B, CAP), jnp.float32),
            jax.ShapeDtypeStruct((B, CAP), jnp.int32),
        ],
        mesh=mesh,
        scratch_types=[
            pltpu.VMEM((v_pad,), jnp.float32),
            pltpu.VMEM((qpw,), jnp.float32),
            pltpu.VMEM((CAP + SC_L,), jnp.float32),
            pltpu.VMEM((CAP + SC_L,), jnp.int32),
        ],
        compiler_params=pltpu.CompilerParams(needs_layout_passes=False),
    )(d2, t.reshape(B))

    fin_chunks = CAP // LANES
    fbody = functools.partial(_finish_body, fin_chunks)
    D, I = pl.pallas_call(
        fbody,
        grid=(B // bb, fin_chunks),
        in_specs=[
            pl.BlockSpec((bb, LANES), lambda i, j: (i, j)),
            pl.BlockSpec((bb, LANES), lambda i, j: (i, j)),
        ],
        out_specs=[
            pl.BlockSpec((bb, K_NB), lambda i, j: (i, 0)),
            pl.BlockSpec((bb, K_NB), lambda i, j: (i, 0)),
        ],
        out_shape=[
            jax.ShapeDtypeStruct((B, K_NB), jnp.float32),
            jax.ShapeDtypeStruct((B, K_NB), jnp.int32),
        ],
        scratch_shapes=[
            pltpu.VMEM((bb, LANES), jnp.float32),
            pltpu.VMEM((bb, LANES), jnp.int32),
        ],
        compiler_params=pltpu.CompilerParams(
            dimension_semantics=("arbitrary", "arbitrary")),
    )(cand_v, cand_i)
    return x, D, I


# speedup vs baseline: 4.9812x; 1.2806x over previous
import functools

import jax
import jax.numpy as jnp
from jax import lax
from jax.experimental import pallas as pl
from jax.experimental.pallas import tpu as pltpu
from jax.experimental.pallas import tpu_sc as plsc

K_NB = 128
LANES = 128
_BIG_IDX = 2**30
SS = 16
CAP = 4096
SC_NC = 2
SC_NS = 16
SC_L = 16


def _lex_less(av, ai, bv, bi):
    return (av < bv) | ((av == bv) & (ai < bi))


def _partner(v, j):
    rp = jnp.roll(v, -j, axis=-1)
    rm = jnp.roll(v, j, axis=-1)
    return rp, rm


def _cmpex(v, i, lane, j, up):
    vp_p, vp_m = _partner(v, j)
    ip_p, ip_m = _partner(i, j)
    lower = (lane & j) == 0
    pv = jnp.where(lower, vp_p, vp_m)
    pi = jnp.where(lower, ip_p, ip_m)
    less = _lex_less(v, i, pv, pi)
    keep_self = (lower == up) == less
    return jnp.where(keep_self, v, pv), jnp.where(keep_self, i, pi)


def _sort_lanes(v, i, lane, ascending):
    for kk in (2, 4, 8, 16, 32, 64, 128):
        blk_up = (lane & kk) == 0
        if not ascending:
            blk_up = jnp.logical_not(blk_up)
        jj = kk // 2
        while jj >= 1:
            v, i = _cmpex(v, i, lane, jj, blk_up)
            jj //= 2
    return v, i


def _merge_desc_into_asc(bufv, bufi, cv, ci, lane):
    less = _lex_less(bufv, bufi, cv, ci)
    nv = jnp.where(less, bufv, cv)
    ni = jnp.where(less, bufi, ci)
    true_up = lane >= 0
    for j in (64, 32, 16, 8, 4, 2, 1):
        nv, ni = _cmpex(nv, ni, lane, j, true_up)
    return nv, ni


def _cmpex_v(v, lane, j, up):
    vp_p, vp_m = _partner(v, j)
    lower = (lane & j) == 0
    pv = jnp.where(lower, vp_p, vp_m)
    keep_self = (lower == up) == (v < pv)
    return jnp.where(keep_self, v, pv)


def _sort_lanes_v(v, lane, ascending):
    for kk in (2, 4, 8, 16, 32, 64, 128):
        blk_up = (lane & kk) == 0
        if not ascending:
            blk_up = jnp.logical_not(blk_up)
        jj = kk // 2
        while jj >= 1:
            v = _cmpex_v(v, lane, jj, blk_up)
            jj //= 2
    return v


def _merge_desc_into_asc_v(bufv, cv, lane):
    nv = jnp.minimum(bufv, cv)
    true_up = lane >= 0
    for j in (64, 32, 16, 8, 4, 2, 1):
        nv = _cmpex_v(nv, lane, j, true_up)
    return nv


def _score_body(n_chunks, v_real, x_ref, w_ref, xsq_ref, wsq_ref,
                d2_ref, t_ref, bufv):
    j = pl.program_id(1)

    @pl.when(j == 0)
    def _init():
        bufv[...] = jnp.full(bufv.shape, jnp.inf, jnp.float32)

    xb = x_ref[...]
    wb = w_ref[...]
    wsq = wsq_ref[0]
    xsq = xsq_ref[...]
    mm = lax.dot_general(
        xb, wb, (((1,), (1,)), ((), ())),
        preferred_element_type=jnp.float32)
    s = (xsq - 2.0 * mm) + wsq

    shape = s.shape
    lane = lax.broadcasted_iota(jnp.int32, shape, 1)
    col = lane + j * LANES
    s = jnp.where(col < v_real, s, jnp.inf)
    d2_ref[...] = s

    @pl.when(j % SS == 0)
    def _merge():
        cv = _sort_lanes_v(s, lane, ascending=False)
        bufv[...] = _merge_desc_into_asc_v(bufv[...], cv, lane)

    @pl.when(j == n_chunks - 1)
    def _fin():
        t_ref[...] = bufv[:, K_NB - 1:K_NB]


def _sc_filter_body(qpw, vpad, d2_hbm, t_hbm, cv_hbm, ci_hbm,
                    row_v, t_v, cvb, cib):
    wid = lax.axis_index("s") * SC_NC + lax.axis_index("c")
    base = wid * qpw
    pltpu.sync_copy(t_hbm.at[pl.ds(base, qpw)], t_v)

    inf16 = jnp.full((SC_L,), jnp.inf, jnp.float32)
    big16 = jnp.full((SC_L,), _BIG_IDX, jnp.int32)
    iota16 = lax.iota(jnp.int32, SC_L)

    def per_query(qi, carry):
        qg = base + qi
        pltpu.sync_copy(d2_hbm.at[qg], row_v)
        tblk = t_v[pl.ds((qi // SC_L) * SC_L, SC_L)]
        tsel = jnp.where(iota16 == qi % SC_L, tblk, -jnp.inf)
        thr = jnp.max(tsel)

        def init_step(n, c):
            cvb[pl.ds(n * SC_L, SC_L)] = inf16
            cib[pl.ds(n * SC_L, SC_L)] = big16
            return c
        lax.fori_loop(0, (CAP + SC_L) // SC_L, init_step, 0)

        def scan_step(n, ptr):
            v = row_v[pl.ds(n * SC_L, SC_L)]
            m = v <= thr
            gi = iota16 + n * SC_L
            plsc.store_compressed(cvb.at[pl.ds(ptr, SC_L)], v, mask=m)
            plsc.store_compressed(cib.at[pl.ds(ptr, SC_L)], gi, mask=m)
            cnt = jnp.max(plsc.all_reduce_population_count(m))
            return jnp.minimum(ptr + cnt, CAP)
        lax.fori_loop(0, vpad // SC_L, scan_step, jnp.int32(0))

        pltpu.sync_copy(cvb.at[pl.ds(0, CAP)], cv_hbm.at[qg])
        pltpu.sync_copy(cib.at[pl.ds(0, CAP)], ci_hbm.at[qg])
        return carry

    lax.fori_loop(0, qpw, per_query, 0)


def _finish_body(n_chunks, cv_ref, ci_ref, d_ref, i_ref, bufv, bufi):
    j = pl.program_id(1)

    @pl.when(j == 0)
    def _init():
        bufv[...] = jnp.full(bufv.shape, jnp.inf, jnp.float32)
        bufi[...] = jnp.full(bufi.shape, _BIG_IDX, jnp.int32)

    v = cv_ref[...]
    idx = ci_ref[...]
    lane = lax.broadcasted_iota(jnp.int32, v.shape, 1)
    cv, ci = _sort_lanes(v, idx, lane, ascending=False)
    nv, ni = _merge_desc_into_asc(bufv[...], bufi[...], cv, ci, lane)
    bufv[...] = nv
    bufi[...] = ni

    @pl.when(j == n_chunks - 1)
    def _fin():
        d_ref[...] = bufv[...]
        i_ref[...] = bufi[...]


def kernel(x, W):
    B, d = x.shape
    V, _ = W.shape
    n_chunks = -(-V // LANES)
    if n_chunks % SS:
        n_chunks += SS - n_chunks % SS
    v_pad = n_chunks * LANES
    xsq = jnp.sum(x * x, axis=1, keepdims=True)
    wsq = jnp.sum(W * W, axis=1)
    if v_pad != V:
        W = jnp.pad(W, ((0, v_pad - V), (0, 0)))
        wsq = jnp.pad(wsq, (0, v_pad - V))
    wsq3 = wsq.reshape(n_chunks, 1, LANES)
    bb = 256
    grid = (B // bb, n_chunks)

    body = functools.partial(_score_body, n_chunks, V)
    d2, t = pl.pallas_call(
        body,
        grid=grid,
        in_specs=[
            pl.BlockSpec((bb, d), lambda i, j: (i, 0)),
            pl.BlockSpec((LANES, d), lambda i, j: (j, 0)),
            pl.BlockSpec((bb, 1), lambda i, j: (i, 0)),
            pl.BlockSpec((1, 1, LANES), lambda i, j: (j, 0, 0)),
        ],
        out_specs=[
            pl.BlockSpec((bb, LANES), lambda i, j: (i, j)),
            pl.BlockSpec((bb, 1), lambda i, j: (i, 0)),
        ],
        out_shape=[
            jax.ShapeDtypeStruct((B, v_pad), jnp.float32),
            jax.ShapeDtypeStruct((B, 1), jnp.float32),
        ],
        scratch_shapes=[
            pltpu.VMEM((bb, LANES), jnp.float32),
        ],
        compiler_params=pltpu.CompilerParams(
            dimension_semantics=("arbitrary", "arbitrary")),
    )(x, W, xsq, wsq3)

    n_workers = SC_NC * SC_NS
    qpw = B // n_workers
    mesh = plsc.VectorSubcoreMesh(core_axis_name="c", subcore_axis_name="s")
    sc_body = functools.partial(_sc_filter_body, qpw, v_pad)
    cand_v, cand_i = pl.kernel(
        sc_body,
        out_type=[
            jax.ShapeDtypeStruct((B, CAP), jnp.float32),
            jax.ShapeDtypeStruct((B, CAP), jnp.int32),
        ],
        mesh=mesh,
        scratch_types=[
            pltpu.VMEM((v_pad,), jnp.float32),
            pltpu.VMEM((qpw,), jnp.float32),
            pltpu.VMEM((CAP + SC_L,), jnp.float32),
            pltpu.VMEM((CAP + SC_L,), jnp.int32),
        ],
        compiler_params=pltpu.CompilerParams(needs_layout_passes=False),
    )(d2, t.reshape(B))

    fin_chunks = CAP // LANES
    fbody = functools.partial(_finish_body, fin_chunks)
    D, I = pl.pallas_call(
        fbody,
        grid=(B // bb, fin_chunks),
        in_specs=[
            pl.BlockSpec((bb, LANES), lambda i, j: (i, j)),
            pl.BlockSpec((bb, LANES), lambda i, j: (i, j)),
        ],
        out_specs=[
            pl.BlockSpec((bb, K_NB), lambda i, j: (i, 0)),
            pl.BlockSpec((bb, K_NB), lambda i, j: (i, 0)),
        ],
        out_shape=[
            jax.ShapeDtypeStruct((B, K_NB), jnp.float32),
            jax.ShapeDtypeStruct((B, K_NB), jnp.int32),
        ],
        scratch_shapes=[
            pltpu.VMEM((bb, LANES), jnp.float32),
            pltpu.VMEM((bb, LANES), jnp.int32),
        ],
        compiler_params=pltpu.CompilerParams(
            dimension_semantics=("arbitrary", "arbitrary")),
    )(cand_v, cand_i)
    return x, D, I


# speedup vs baseline: 6.3287x; 1.2705x over previous
import functools

import jax
import jax.numpy as jnp
from jax import lax
from jax.experimental import pallas as pl
from jax.experimental.pallas import tpu as pltpu
from jax.experimental.pallas import tpu_sc as plsc

K_NB = 128
LANES = 128
_BIG_IDX = 2**30
SS = 16
CAP = 4096
SC_NC = 2
SC_NS = 16
SC_L = 16
SC_G = 8


def _lex_less(av, ai, bv, bi):
    return (av < bv) | ((av == bv) & (ai < bi))


def _partner(v, j):
    rp = jnp.roll(v, -j, axis=-1)
    rm = jnp.roll(v, j, axis=-1)
    return rp, rm


def _cmpex(v, i, lane, j, up):
    vp_p, vp_m = _partner(v, j)
    ip_p, ip_m = _partner(i, j)
    lower = (lane & j) == 0
    pv = jnp.where(lower, vp_p, vp_m)
    pi = jnp.where(lower, ip_p, ip_m)
    less = _lex_less(v, i, pv, pi)
    keep_self = (lower == up) == less
    return jnp.where(keep_self, v, pv), jnp.where(keep_self, i, pi)


def _sort_lanes(v, i, lane, ascending):
    for kk in (2, 4, 8, 16, 32, 64, 128):
        blk_up = (lane & kk) == 0
        if not ascending:
            blk_up = jnp.logical_not(blk_up)
        jj = kk // 2
        while jj >= 1:
            v, i = _cmpex(v, i, lane, jj, blk_up)
            jj //= 2
    return v, i


def _merge_desc_into_asc(bufv, bufi, cv, ci, lane):
    less = _lex_less(bufv, bufi, cv, ci)
    nv = jnp.where(less, bufv, cv)
    ni = jnp.where(less, bufi, ci)
    true_up = lane >= 0
    for j in (64, 32, 16, 8, 4, 2, 1):
        nv, ni = _cmpex(nv, ni, lane, j, true_up)
    return nv, ni


def _cmpex_v(v, lane, j, up):
    vp_p, vp_m = _partner(v, j)
    lower = (lane & j) == 0
    pv = jnp.where(lower, vp_p, vp_m)
    keep_self = (lower == up) == (v < pv)
    return jnp.where(keep_self, v, pv)


def _sort_lanes_v(v, lane, ascending):
    for kk in (2, 4, 8, 16, 32, 64, 128):
        blk_up = (lane & kk) == 0
        if not ascending:
            blk_up = jnp.logical_not(blk_up)
        jj = kk // 2
        while jj >= 1:
            v = _cmpex_v(v, lane, jj, blk_up)
            jj //= 2
    return v


def _merge_desc_into_asc_v(bufv, cv, lane):
    nv = jnp.minimum(bufv, cv)
    true_up = lane >= 0
    for j in (64, 32, 16, 8, 4, 2, 1):
        nv = _cmpex_v(nv, lane, j, true_up)
    return nv


def _score_body(n_chunks, v_real, x_ref, w_ref, xsq_ref, wsq_ref,
                d2_ref, t_ref, bufv):
    j = pl.program_id(1)

    @pl.when(j == 0)
    def _init():
        bufv[...] = jnp.full(bufv.shape, jnp.inf, jnp.float32)

    xb = x_ref[...]
    wb = w_ref[...]
    wsq = wsq_ref[0]
    xsq = xsq_ref[...]
    mm = lax.dot_general(
        xb, wb, (((1,), (1,)), ((), ())),
        preferred_element_type=jnp.float32)
    s = (xsq - 2.0 * mm) + wsq

    shape = s.shape
    lane = lax.broadcasted_iota(jnp.int32, shape, 1)
    col = lane + j * LANES
    s = jnp.where(col < v_real, s, jnp.inf)
    d2_ref[...] = s

    @pl.when(j % SS == 0)
    def _merge():
        cv = _sort_lanes_v(s, lane, ascending=False)
        bufv[...] = _merge_desc_into_asc_v(bufv[...], cv, lane)

    @pl.when(j == n_chunks - 1)
    def _fin():
        t_ref[...] = bufv[:, K_NB - 1:K_NB]


def _sc_filter_body(qpw, vpad, d2_hbm, t_hbm, cv_hbm, ci_hbm,
                    row_v, t_v, cvb, cib):
    wid = lax.axis_index("s") * SC_NC + lax.axis_index("c")
    base = wid * qpw
    pltpu.sync_copy(t_hbm.at[pl.ds(base, qpw)], t_v)

    inf16 = jnp.full((SC_L,), jnp.inf, jnp.float32)
    big16 = jnp.full((SC_L,), _BIG_IDX, jnp.int32)
    iota16 = lax.iota(jnp.int32, SC_L)

    def per_query(qi, carry):
        qg = base + qi
        pltpu.sync_copy(d2_hbm.at[qg], row_v)
        tblk = t_v[pl.ds((qi // SC_L) * SC_L, SC_L)]
        tsel = jnp.where(iota16 == qi % SC_L, tblk, -jnp.inf)
        thr = jnp.max(tsel)

        def init_step(n, c):
            cvb[pl.ds(n * SC_L, SC_L)] = inf16
            cib[pl.ds(n * SC_L, SC_L)] = big16
            return c
        lax.fori_loop(0, (CAP + SC_L) // SC_L, init_step, 0)

        def scan_group(g, ptr):
            vs, ms, cnts = [], [], []
            for u in range(SC_G):
                v = row_v[pl.ds((g * SC_G + u) * SC_L, SC_L)]
                m = v <= thr
                vs.append(v)
                ms.append(m)
                cnts.append(jnp.max(plsc.all_reduce_population_count(m)))
            for u in range(SC_G):
                gi = iota16 + (g * SC_G + u) * SC_L
                plsc.store_compressed(cvb.at[pl.ds(ptr, SC_L)], vs[u],
                                      mask=ms[u])
                plsc.store_compressed(cib.at[pl.ds(ptr, SC_L)], gi,
                                      mask=ms[u])
                ptr = jnp.minimum(ptr + cnts[u], CAP)
            return ptr
        lax.fori_loop(0, vpad // (SC_L * SC_G), scan_group, jnp.int32(0))

        pltpu.sync_copy(cvb.at[pl.ds(0, CAP)], cv_hbm.at[qg])
        pltpu.sync_copy(cib.at[pl.ds(0, CAP)], ci_hbm.at[qg])
        return carry

    lax.fori_loop(0, qpw, per_query, 0)


def _finish_body(n_chunks, cv_ref, ci_ref, d_ref, i_ref, bufv, bufi):
    j = pl.program_id(1)

    @pl.when(j == 0)
    def _init():
        bufv[...] = jnp.full(bufv.shape, jnp.inf, jnp.float32)
        bufi[...] = jnp.full(bufi.shape, _BIG_IDX, jnp.int32)

    v = cv_ref[...]
    idx = ci_ref[...]
    lane = lax.broadcasted_iota(jnp.int32, v.shape, 1)
    cv, ci = _sort_lanes(v, idx, lane, ascending=False)
    nv, ni = _merge_desc_into_asc(bufv[...], bufi[...], cv, ci, lane)
    bufv[...] = nv
    bufi[...] = ni

    @pl.when(j == n_chunks - 1)
    def _fin():
        d_ref[...] = bufv[...]
        i_ref[...] = bufi[...]


def kernel(x, W):
    B, d = x.shape
    V, _ = W.shape
    n_chunks = -(-V // LANES)
    if n_chunks % SS:
        n_chunks += SS - n_chunks % SS
    v_pad = n_chunks * LANES
    xsq = jnp.sum(x * x, axis=1, keepdims=True)
    wsq = jnp.sum(W * W, axis=1)
    if v_pad != V:
        W = jnp.pad(W, ((0, v_pad - V), (0, 0)))
        wsq = jnp.pad(wsq, (0, v_pad - V))
    wsq3 = wsq.reshape(n_chunks, 1, LANES)
    bb = 256
    grid = (B // bb, n_chunks)

    body = functools.partial(_score_body, n_chunks, V)
    d2, t = pl.pallas_call(
        body,
        grid=grid,
        in_specs=[
            pl.BlockSpec((bb, d), lambda i, j: (i, 0)),
            pl.BlockSpec((LANES, d), lambda i, j: (j, 0)),
            pl.BlockSpec((bb, 1), lambda i, j: (i, 0)),
            pl.BlockSpec((1, 1, LANES), lambda i, j: (j, 0, 0)),
        ],
        out_specs=[
            pl.BlockSpec((bb, LANES), lambda i, j: (i, j)),
            pl.BlockSpec((bb, 1), lambda i, j: (i, 0)),
        ],
        out_shape=[
            jax.ShapeDtypeStruct((B, v_pad), jnp.float32),
            jax.ShapeDtypeStruct((B, 1), jnp.float32),
        ],
        scratch_shapes=[
            pltpu.VMEM((bb, LANES), jnp.float32),
        ],
        compiler_params=pltpu.CompilerParams(
            dimension_semantics=("arbitrary", "arbitrary")),
    )(x, W, xsq, wsq3)

    n_workers = SC_NC * SC_NS
    qpw = B // n_workers
    mesh = plsc.VectorSubcoreMesh(core_axis_name="c", subcore_axis_name="s")
    sc_body = functools.partial(_sc_filter_body, qpw, v_pad)
    cand_v, cand_i = pl.kernel(
        sc_body,
        out_type=[
            jax.ShapeDtypeStruct((B, CAP), jnp.float32),
            jax.ShapeDtypeStruct((B, CAP), jnp.int32),
        ],
        mesh=mesh,
        scratch_types=[
            pltpu.VMEM((v_pad,), jnp.float32),
            pltpu.VMEM((qpw,), jnp.float32),
            pltpu.VMEM((CAP + SC_L,), jnp.float32),
            pltpu.VMEM((CAP + SC_L,), jnp.int32),
        ],
        compiler_params=pltpu.CompilerParams(needs_layout_passes=False),
    )(d2, t.reshape(B))

    fin_chunks = CAP // LANES
    fbody = functools.partial(_finish_body, fin_chunks)
    D, I = pl.pallas_call(
        fbody,
        grid=(B // bb, fin_chunks),
        in_specs=[
            pl.BlockSpec((bb, LANES), lambda i, j: (i, j)),
            pl.BlockSpec((bb, LANES), lambda i, j: (i, j)),
        ],
        out_specs=[
            pl.BlockSpec((bb, K_NB), lambda i, j: (i, 0)),
            pl.BlockSpec((bb, K_NB), lambda i, j: (i, 0)),
        ],
        out_shape=[
            jax.ShapeDtypeStruct((B, K_NB), jnp.float32),
            jax.ShapeDtypeStruct((B, K_NB), jnp.int32),
        ],
        scratch_shapes=[
            pltpu.VMEM((bb, LANES), jnp.float32),
            pltpu.VMEM((bb, LANES), jnp.int32),
        ],
        compiler_params=pltpu.CompilerParams(
            dimension_semantics=("arbitrary", "arbitrary")),
    )(cand_v, cand_i)
    return x, D, I


# speedup vs baseline: 8.0490x; 1.2718x over previous
import functools

import jax
import jax.numpy as jnp
from jax import lax
from jax.experimental import pallas as pl
from jax.experimental.pallas import tpu as pltpu
from jax.experimental.pallas import tpu_sc as plsc

K_NB = 128
LANES = 128
_BIG_IDX = 2**30
SS = 16
CAP = 4096
SC_NC = 2
SC_NS = 16
SC_L = 16
SC_G = 8


def _lex_less(av, ai, bv, bi):
    return (av < bv) | ((av == bv) & (ai < bi))


def _partner(v, j):
    rp = jnp.roll(v, -j, axis=-1)
    rm = jnp.roll(v, j, axis=-1)
    return rp, rm


def _cmpex(v, i, lane, j, up):
    vp_p, vp_m = _partner(v, j)
    ip_p, ip_m = _partner(i, j)
    lower = (lane & j) == 0
    pv = jnp.where(lower, vp_p, vp_m)
    pi = jnp.where(lower, ip_p, ip_m)
    less = _lex_less(v, i, pv, pi)
    keep_self = (lower == up) == less
    return jnp.where(keep_self, v, pv), jnp.where(keep_self, i, pi)


def _sort_lanes(v, i, lane, ascending):
    for kk in (2, 4, 8, 16, 32, 64, 128):
        blk_up = (lane & kk) == 0
        if not ascending:
            blk_up = jnp.logical_not(blk_up)
        jj = kk // 2
        while jj >= 1:
            v, i = _cmpex(v, i, lane, jj, blk_up)
            jj //= 2
    return v, i


def _merge_desc_into_asc(bufv, bufi, cv, ci, lane):
    less = _lex_less(bufv, bufi, cv, ci)
    nv = jnp.where(less, bufv, cv)
    ni = jnp.where(less, bufi, ci)
    true_up = lane >= 0
    for j in (64, 32, 16, 8, 4, 2, 1):
        nv, ni = _cmpex(nv, ni, lane, j, true_up)
    return nv, ni


def _cmpex_v(v, lane, j, up):
    vp_p, vp_m = _partner(v, j)
    lower = (lane & j) == 0
    pv = jnp.where(lower, vp_p, vp_m)
    keep_self = (lower == up) == (v < pv)
    return jnp.where(keep_self, v, pv)


def _sort_lanes_v(v, lane, ascending):
    for kk in (2, 4, 8, 16, 32, 64, 128):
        blk_up = (lane & kk) == 0
        if not ascending:
            blk_up = jnp.logical_not(blk_up)
        jj = kk // 2
        while jj >= 1:
            v = _cmpex_v(v, lane, jj, blk_up)
            jj //= 2
    return v


def _merge_desc_into_asc_v(bufv, cv, lane):
    nv = jnp.minimum(bufv, cv)
    true_up = lane >= 0
    for j in (64, 32, 16, 8, 4, 2, 1):
        nv = _cmpex_v(nv, lane, j, true_up)
    return nv


def _score_body(n_chunks, v_real, x_ref, w_ref, xsq_ref, wsq_ref,
                d2_ref, t_ref, bufv):
    j = pl.program_id(1)

    @pl.when(j == 0)
    def _init():
        bufv[...] = jnp.full(bufv.shape, jnp.inf, jnp.float32)

    xb = x_ref[...]
    wb = w_ref[...]
    wsq = wsq_ref[0]
    xsq = xsq_ref[...]
    mm = lax.dot_general(
        xb, wb, (((1,), (1,)), ((), ())),
        preferred_element_type=jnp.float32)
    s = (xsq - 2.0 * mm) + wsq

    shape = s.shape
    lane = lax.broadcasted_iota(jnp.int32, shape, 1)
    col = lane + j * LANES
    s = jnp.where(col < v_real, s, jnp.inf)
    d2_ref[...] = s

    @pl.when(j % SS == 0)
    def _merge():
        cv = _sort_lanes_v(s, lane, ascending=False)
        bufv[...] = _merge_desc_into_asc_v(bufv[...], cv, lane)

    @pl.when(j == n_chunks - 1)
    def _fin():
        t_ref[...] = bufv[:, K_NB - 1:K_NB]


def _sc_filter_body(qpw, vpad, d2_hbm, t_hbm, cv_hbm, ci_hbm,
                    row_v, t_v, cvb, cib):
    wid = lax.axis_index("s") * SC_NC + lax.axis_index("c")
    base = wid * qpw
    pltpu.sync_copy(t_hbm.at[pl.ds(base, qpw)], t_v)

    inf16 = jnp.full((SC_L,), jnp.inf, jnp.float32)
    big16 = jnp.full((SC_L,), _BIG_IDX, jnp.int32)
    iota16 = lax.iota(jnp.int32, SC_L)

    def per_query(qi, carry):
        qg = base + qi
        pltpu.sync_copy(d2_hbm.at[qg], row_v)
        tblk = t_v[pl.ds((qi // SC_L) * SC_L, SC_L)]
        tsel = jnp.where(iota16 == qi % SC_L, tblk, -jnp.inf)
        thr = jnp.max(tsel)

        def init_step(n, c):
            cvb[pl.ds(n * SC_L, SC_L)] = inf16
            cib[pl.ds(n * SC_L, SC_L)] = big16
            return c
        lax.fori_loop(0, (CAP + SC_L) // SC_L, init_step, 0)

        def scan_group(g, ptr):
            vs, ms, cnts = [], [], []
            for u in range(SC_G):
                v = row_v[pl.ds((g * SC_G + u) * SC_L, SC_L)]
                m = v <= thr
                vs.append(v)
                ms.append(m)
                cnts.append(jnp.max(plsc.all_reduce_population_count(m)))
            for u in range(SC_G):
                gi = iota16 + (g * SC_G + u) * SC_L
                plsc.store_compressed(cvb.at[pl.ds(ptr, SC_L)], vs[u],
                                      mask=ms[u])
                plsc.store_compressed(cib.at[pl.ds(ptr, SC_L)], gi,
                                      mask=ms[u])
                ptr = jnp.minimum(ptr + cnts[u], CAP)
            return ptr
        lax.fori_loop(0, vpad // (SC_L * SC_G), scan_group, jnp.int32(0))

        pltpu.sync_copy(cvb.at[pl.ds(0, CAP)], cv_hbm.at[qg])
        pltpu.sync_copy(cib.at[pl.ds(0, CAP)], ci_hbm.at[qg])
        return carry

    lax.fori_loop(0, qpw, per_query, 0)


def _finish_body(n_chunks, cv_ref, ci_ref, d_ref, i_ref, bufv, bufi):
    j = pl.program_id(1)

    @pl.when(j == 0)
    def _init():
        bufv[...] = jnp.full(bufv.shape, jnp.inf, jnp.float32)
        bufi[...] = jnp.full(bufi.shape, _BIG_IDX, jnp.int32)

    v = cv_ref[...]
    idx = ci_ref[...]
    lane = lax.broadcasted_iota(jnp.int32, v.shape, 1)
    cv, ci = _sort_lanes(v, idx, lane, ascending=False)
    nv, ni = _merge_desc_into_asc(bufv[...], bufi[...], cv, ci, lane)
    bufv[...] = nv
    bufi[...] = ni

    @pl.when(j == n_chunks - 1)
    def _fin():
        d_ref[...] = bufv[...]
        i_ref[...] = bufi[...]


NH = 2


def kernel(x, W):
    B, d = x.shape
    V, _ = W.shape
    n_chunks = -(-V // LANES)
    if n_chunks % SS:
        n_chunks += SS - n_chunks % SS
    v_pad = n_chunks * LANES
    xsq = jnp.sum(x * x, axis=1, keepdims=True)
    wsq = jnp.sum(W * W, axis=1)
    if v_pad != V:
        W = jnp.pad(W, ((0, v_pad - V), (0, 0)))
        wsq = jnp.pad(wsq, (0, v_pad - V))
    wsq3 = wsq.reshape(n_chunks, 1, LANES)
    bb = 256
    hb = B // NH

    body = functools.partial(_score_body, n_chunks, V)
    n_workers = SC_NC * SC_NS
    qpw = hb // n_workers
    mesh = plsc.VectorSubcoreMesh(core_axis_name="c", subcore_axis_name="s")
    sc_body = functools.partial(_sc_filter_body, qpw, v_pad)
    fin_chunks = CAP // LANES
    fbody = functools.partial(_finish_body, fin_chunks)

    ds, is_ = [], []
    for h in range(NH):
        xh = x[h * hb:(h + 1) * hb]
        xsqh = xsq[h * hb:(h + 1) * hb]
        d2, t = pl.pallas_call(
            body,
            grid=(hb // bb, n_chunks),
            in_specs=[
                pl.BlockSpec((bb, d), lambda i, j: (i, 0)),
                pl.BlockSpec((LANES, d), lambda i, j: (j, 0)),
                pl.BlockSpec((bb, 1), lambda i, j: (i, 0)),
                pl.BlockSpec((1, 1, LANES), lambda i, j: (j, 0, 0)),
            ],
            out_specs=[
                pl.BlockSpec((bb, LANES), lambda i, j: (i, j)),
                pl.BlockSpec((bb, 1), lambda i, j: (i, 0)),
            ],
            out_shape=[
                jax.ShapeDtypeStruct((hb, v_pad), jnp.float32),
                jax.ShapeDtypeStruct((hb, 1), jnp.float32),
            ],
            scratch_shapes=[
                pltpu.VMEM((bb, LANES), jnp.float32),
            ],
            compiler_params=pltpu.CompilerParams(
                dimension_semantics=("arbitrary", "arbitrary")),
        )(xh, W, xsqh, wsq3)

        cand_v, cand_i = pl.kernel(
            sc_body,
            out_type=[
                jax.ShapeDtypeStruct((hb, CAP), jnp.float32),
                jax.ShapeDtypeStruct((hb, CAP), jnp.int32),
            ],
            mesh=mesh,
            scratch_types=[
                pltpu.VMEM((v_pad,), jnp.float32),
                pltpu.VMEM((qpw,), jnp.float32),
                pltpu.VMEM((CAP + SC_L,), jnp.float32),
                pltpu.VMEM((CAP + SC_L,), jnp.int32),
            ],
            compiler_params=pltpu.CompilerParams(needs_layout_passes=False),
        )(d2, t.reshape(hb))

        D, I = pl.pallas_call(
            fbody,
            grid=(hb // bb, fin_chunks),
            in_specs=[
                pl.BlockSpec((bb, LANES), lambda i, j: (i, j)),
                pl.BlockSpec((bb, LANES), lambda i, j: (i, j)),
            ],
            out_specs=[
                pl.BlockSpec((bb, K_NB), lambda i, j: (i, 0)),
                pl.BlockSpec((bb, K_NB), lambda i, j: (i, 0)),
            ],
            out_shape=[
                jax.ShapeDtypeStruct((hb, K_NB), jnp.float32),
                jax.ShapeDtypeStruct((hb, K_NB), jnp.int32),
            ],
            scratch_shapes=[
                pltpu.VMEM((bb, LANES), jnp.float32),
                pltpu.VMEM((bb, LANES), jnp.int32),
            ],
            compiler_params=pltpu.CompilerParams(
                dimension_semantics=("arbitrary", "arbitrary")),
        )(cand_v, cand_i)
        ds.append(D)
        is_.append(I)

    return x, jnp.concatenate(ds, axis=0), jnp.concatenate(is_, axis=0)


# speedup vs baseline: 8.1993x; 1.0187x over previous
import functools

import jax
import jax.numpy as jnp
from jax import lax
from jax.experimental import pallas as pl
from jax.experimental.pallas import tpu as pltpu
from jax.experimental.pallas import tpu_sc as plsc

K_NB = 128
LANES = 128
_BIG_IDX = 2**30
SS = 16
CAP = 4096
SC_NC = 2
SC_NS = 16
SC_L = 16
SC_G = 8


def _lex_less(av, ai, bv, bi):
    return (av < bv) | ((av == bv) & (ai < bi))


def _partner(v, j):
    rp = jnp.roll(v, -j, axis=-1)
    rm = jnp.roll(v, j, axis=-1)
    return rp, rm


def _cmpex(v, i, lane, j, up):
    vp_p, vp_m = _partner(v, j)
    ip_p, ip_m = _partner(i, j)
    lower = (lane & j) == 0
    pv = jnp.where(lower, vp_p, vp_m)
    pi = jnp.where(lower, ip_p, ip_m)
    less = _lex_less(v, i, pv, pi)
    keep_self = (lower == up) == less
    return jnp.where(keep_self, v, pv), jnp.where(keep_self, i, pi)


def _sort_lanes(v, i, lane, ascending):
    for kk in (2, 4, 8, 16, 32, 64, 128):
        blk_up = (lane & kk) == 0
        if not ascending:
            blk_up = jnp.logical_not(blk_up)
        jj = kk // 2
        while jj >= 1:
            v, i = _cmpex(v, i, lane, jj, blk_up)
            jj //= 2
    return v, i


def _merge_desc_into_asc(bufv, bufi, cv, ci, lane):
    less = _lex_less(bufv, bufi, cv, ci)
    nv = jnp.where(less, bufv, cv)
    ni = jnp.where(less, bufi, ci)
    true_up = lane >= 0
    for j in (64, 32, 16, 8, 4, 2, 1):
        nv, ni = _cmpex(nv, ni, lane, j, true_up)
    return nv, ni


def _cmpex_v(v, lane, j, up):
    vp_p, vp_m = _partner(v, j)
    lower = (lane & j) == 0
    pv = jnp.where(lower, vp_p, vp_m)
    keep_self = (lower == up) == (v < pv)
    return jnp.where(keep_self, v, pv)


def _sort_lanes_v(v, lane, ascending):
    for kk in (2, 4, 8, 16, 32, 64, 128):
        blk_up = (lane & kk) == 0
        if not ascending:
            blk_up = jnp.logical_not(blk_up)
        jj = kk // 2
        while jj >= 1:
            v = _cmpex_v(v, lane, jj, blk_up)
            jj //= 2
    return v


def _merge_desc_into_asc_v(bufv, cv, lane):
    nv = jnp.minimum(bufv, cv)
    true_up = lane >= 0
    for j in (64, 32, 16, 8, 4, 2, 1):
        nv = _cmpex_v(nv, lane, j, true_up)
    return nv


def _score_body(n_chunks, v_real, x_ref, w_ref, xsq_ref, wsq_ref,
                d2_ref, t_ref, bufv):
    j = pl.program_id(1)

    @pl.when(j == 0)
    def _init():
        bufv[...] = jnp.full(bufv.shape, jnp.inf, jnp.float32)

    xb = x_ref[...]
    wb = w_ref[...]
    wsq = wsq_ref[0]
    xsq = xsq_ref[...]
    mm = lax.dot_general(
        xb, wb, (((1,), (1,)), ((), ())),
        preferred_element_type=jnp.float32)
    s = (xsq - 2.0 * mm) + wsq

    shape = s.shape
    lane = lax.broadcasted_iota(jnp.int32, shape, 1)
    col = lane + j * LANES
    s = jnp.where(col < v_real, s, jnp.inf)
    d2_ref[...] = s

    @pl.when(j % SS == 0)
    def _merge():
        cv = _sort_lanes_v(s, lane, ascending=False)
        bufv[...] = _merge_desc_into_asc_v(bufv[...], cv, lane)

    @pl.when(j == n_chunks - 1)
    def _fin():
        t_ref[...] = bufv[:, K_NB - 1:K_NB]


def _sc_filter_body(qpw, vpad, d2_hbm, t_hbm, cv_hbm, ci_hbm,
                    row_v, t_v, cvb, cib):
    wid = lax.axis_index("s") * SC_NC + lax.axis_index("c")
    base = wid * qpw
    pltpu.sync_copy(t_hbm.at[pl.ds(base, qpw)], t_v)

    inf16 = jnp.full((SC_L,), jnp.inf, jnp.float32)
    big16 = jnp.full((SC_L,), _BIG_IDX, jnp.int32)
    iota16 = lax.iota(jnp.int32, SC_L)

    def per_query(qi, carry):
        qg = base + qi
        pltpu.sync_copy(d2_hbm.at[qg], row_v)
        tblk = t_v[pl.ds((qi // SC_L) * SC_L, SC_L)]
        tsel = jnp.where(iota16 == qi % SC_L, tblk, -jnp.inf)
        thr = jnp.max(tsel)

        def init_step(n, c):
            cvb[pl.ds(n * SC_L, SC_L)] = inf16
            cib[pl.ds(n * SC_L, SC_L)] = big16
            return c
        lax.fori_loop(0, (CAP + SC_L) // SC_L, init_step, 0)

        def scan_group(g, ptr):
            vs, ms, cnts = [], [], []
            for u in range(SC_G):
                v = row_v[pl.ds((g * SC_G + u) * SC_L, SC_L)]
                m = v <= thr
                vs.append(v)
                ms.append(m)
                cnts.append(jnp.max(plsc.all_reduce_population_count(m)))
            for u in range(SC_G):
                gi = iota16 + (g * SC_G + u) * SC_L
                plsc.store_compressed(cvb.at[pl.ds(ptr, SC_L)], vs[u],
                                      mask=ms[u])
                plsc.store_compressed(cib.at[pl.ds(ptr, SC_L)], gi,
                                      mask=ms[u])
                ptr = jnp.minimum(ptr + cnts[u], CAP)
            return ptr
        lax.fori_loop(0, vpad // (SC_L * SC_G), scan_group, jnp.int32(0))

        pltpu.sync_copy(cvb.at[pl.ds(0, CAP)], cv_hbm.at[qg])
        pltpu.sync_copy(cib.at[pl.ds(0, CAP)], ci_hbm.at[qg])
        return carry

    lax.fori_loop(0, qpw, per_query, 0)


def _finish_body(n_chunks, cv_ref, ci_ref, d_ref, i_ref, bufv, bufi):
    j = pl.program_id(1)

    @pl.when(j == 0)
    def _init():
        bufv[...] = jnp.full(bufv.shape, jnp.inf, jnp.float32)
        bufi[...] = jnp.full(bufi.shape, _BIG_IDX, jnp.int32)

    v = cv_ref[...]
    idx = ci_ref[...]
    lane = lax.broadcasted_iota(jnp.int32, v.shape, 1)
    cv, ci = _sort_lanes(v, idx, lane, ascending=False)
    nv, ni = _merge_desc_into_asc(bufv[...], bufi[...], cv, ci, lane)
    bufv[...] = nv
    bufi[...] = ni

    @pl.when(j == n_chunks - 1)
    def _fin():
        d_ref[...] = bufv[...]
        i_ref[...] = bufi[...]


NH = 4


def kernel(x, W):
    B, d = x.shape
    V, _ = W.shape
    n_chunks = -(-V // LANES)
    if n_chunks % SS:
        n_chunks += SS - n_chunks % SS
    v_pad = n_chunks * LANES
    xsq = jnp.sum(x * x, axis=1, keepdims=True)
    wsq = jnp.sum(W * W, axis=1)
    if v_pad != V:
        W = jnp.pad(W, ((0, v_pad - V), (0, 0)))
        wsq = jnp.pad(wsq, (0, v_pad - V))
    wsq3 = wsq.reshape(n_chunks, 1, LANES)
    bb = 256
    hb = B // NH

    body = functools.partial(_score_body, n_chunks, V)
    n_workers = SC_NC * SC_NS
    qpw = hb // n_workers
    mesh = plsc.VectorSubcoreMesh(core_axis_name="c", subcore_axis_name="s")
    sc_body = functools.partial(_sc_filter_body, qpw, v_pad)
    fin_chunks = CAP // LANES
    fbody = functools.partial(_finish_body, fin_chunks)

    ds, is_ = [], []
    for h in range(NH):
        xh = x[h * hb:(h + 1) * hb]
        xsqh = xsq[h * hb:(h + 1) * hb]
        d2, t = pl.pallas_call(
            body,
            grid=(hb // bb, n_chunks),
            in_specs=[
                pl.BlockSpec((bb, d), lambda i, j: (i, 0)),
                pl.BlockSpec((LANES, d), lambda i, j: (j, 0)),
                pl.BlockSpec((bb, 1), lambda i, j: (i, 0)),
                pl.BlockSpec((1, 1, LANES), lambda i, j: (j, 0, 0)),
            ],
            out_specs=[
                pl.BlockSpec((bb, LANES), lambda i, j: (i, j)),
                pl.BlockSpec((bb, 1), lambda i, j: (i, 0)),
            ],
            out_shape=[
                jax.ShapeDtypeStruct((hb, v_pad), jnp.float32),
                jax.ShapeDtypeStruct((hb, 1), jnp.float32),
            ],
            scratch_shapes=[
                pltpu.VMEM((bb, LANES), jnp.float32),
            ],
            compiler_params=pltpu.CompilerParams(
                dimension_semantics=("arbitrary", "arbitrary")),
        )(xh, W, xsqh, wsq3)

        cand_v, cand_i = pl.kernel(
            sc_body,
            out_type=[
                jax.ShapeDtypeStruct((hb, CAP), jnp.float32),
                jax.ShapeDtypeStruct((hb, CAP), jnp.int32),
            ],
            mesh=mesh,
            scratch_types=[
                pltpu.VMEM((v_pad,), jnp.float32),
                pltpu.VMEM((qpw,), jnp.float32),
                pltpu.VMEM((CAP + SC_L,), jnp.float32),
                pltpu.VMEM((CAP + SC_L,), jnp.int32),
            ],
            compiler_params=pltpu.CompilerParams(needs_layout_passes=False),
        )(d2, t.reshape(hb))

        D, I = pl.pallas_call(
            fbody,
            grid=(hb // bb, fin_chunks),
            in_specs=[
                pl.BlockSpec((bb, LANES), lambda i, j: (i, j)),
                pl.BlockSpec((bb, LANES), lambda i, j: (i, j)),
            ],
            out_specs=[
                pl.BlockSpec((bb, K_NB), lambda i, j: (i, 0)),
                pl.BlockSpec((bb, K_NB), lambda i, j: (i, 0)),
            ],
            out_shape=[
                jax.ShapeDtypeStruct((hb, K_NB), jnp.float32),
                jax.ShapeDtypeStruct((hb, K_NB), jnp.int32),
            ],
            scratch_shapes=[
                pltpu.VMEM((bb, LANES), jnp.float32),
                pltpu.VMEM((bb, LANES), jnp.int32),
            ],
            compiler_params=pltpu.CompilerParams(
                dimension_semantics=("arbitrary", "arbitrary")),
        )(cand_v, cand_i)
        ds.append(D)
        is_.append(I)

    return x, jnp.concatenate(ds, axis=0), jnp.concatenate(is_, axis=0)


# speedup vs baseline: 8.9814x; 1.0954x over previous
import functools

import jax
import jax.numpy as jnp
from jax import lax
from jax.experimental import pallas as pl
from jax.experimental.pallas import tpu as pltpu
from jax.experimental.pallas import tpu_sc as plsc

K_NB = 128
LANES = 128
_BIG_IDX = 2**30
SS = 16
CAP = 4096
SC_NC = 2
SC_NS = 16
SC_L = 16
SC_G = 8


def _lex_less(av, ai, bv, bi):
    return (av < bv) | ((av == bv) & (ai < bi))


def _partner(v, j):
    rp = jnp.roll(v, -j, axis=-1)
    rm = jnp.roll(v, j, axis=-1)
    return rp, rm


def _cmpex(v, i, lane, j, up):
    vp_p, vp_m = _partner(v, j)
    ip_p, ip_m = _partner(i, j)
    lower = (lane & j) == 0
    pv = jnp.where(lower, vp_p, vp_m)
    pi = jnp.where(lower, ip_p, ip_m)
    less = _lex_less(v, i, pv, pi)
    keep_self = (lower == up) == less
    return jnp.where(keep_self, v, pv), jnp.where(keep_self, i, pi)


def _sort_lanes(v, i, lane, ascending):
    for kk in (2, 4, 8, 16, 32, 64, 128):
        blk_up = (lane & kk) == 0
        if not ascending:
            blk_up = jnp.logical_not(blk_up)
        jj = kk // 2
        while jj >= 1:
            v, i = _cmpex(v, i, lane, jj, blk_up)
            jj //= 2
    return v, i


def _merge_desc_into_asc(bufv, bufi, cv, ci, lane):
    less = _lex_less(bufv, bufi, cv, ci)
    nv = jnp.where(less, bufv, cv)
    ni = jnp.where(less, bufi, ci)
    true_up = lane >= 0
    for j in (64, 32, 16, 8, 4, 2, 1):
        nv, ni = _cmpex(nv, ni, lane, j, true_up)
    return nv, ni


def _cmpex_v(v, lane, j, up):
    vp_p, vp_m = _partner(v, j)
    lower = (lane & j) == 0
    pv = jnp.where(lower, vp_p, vp_m)
    keep_self = (lower == up) == (v < pv)
    return jnp.where(keep_self, v, pv)


def _sort_lanes_v(v, lane, ascending):
    for kk in (2, 4, 8, 16, 32, 64, 128):
        blk_up = (lane & kk) == 0
        if not ascending:
            blk_up = jnp.logical_not(blk_up)
        jj = kk // 2
        while jj >= 1:
            v = _cmpex_v(v, lane, jj, blk_up)
            jj //= 2
    return v


def _merge_desc_into_asc_v(bufv, cv, lane):
    nv = jnp.minimum(bufv, cv)
    true_up = lane >= 0
    for j in (64, 32, 16, 8, 4, 2, 1):
        nv = _cmpex_v(nv, lane, j, true_up)
    return nv


def _score_body(n_chunks, v_real, x_ref, w_ref, xsq_ref, wsq_ref,
                d2_ref, t_ref, bufv):
    j = pl.program_id(1)

    @pl.when(j == 0)
    def _init():
        bufv[...] = jnp.full(bufv.shape, jnp.inf, jnp.float32)

    xb = x_ref[...]
    wb = w_ref[...]
    wsq = wsq_ref[0]
    xsq = xsq_ref[...]
    mm = lax.dot_general(
        xb, wb, (((1,), (1,)), ((), ())),
        preferred_element_type=jnp.float32)
    s = (xsq - 2.0 * mm) + wsq

    shape = s.shape
    lane = lax.broadcasted_iota(jnp.int32, shape, 1)
    col = lane + j * LANES
    s = jnp.where(col < v_real, s, jnp.inf)
    d2_ref[...] = s

    @pl.when(j % SS == 0)
    def _merge():
        cv = _sort_lanes_v(s, lane, ascending=False)
        bufv[...] = _merge_desc_into_asc_v(bufv[...], cv, lane)

    @pl.when(j == n_chunks - 1)
    def _fin():
        t_ref[...] = bufv[:, K_NB - 1:K_NB]


def _sc_filter_body(qpw, vpad, d2_hbm, t_hbm, cv_hbm, ci_hbm,
                    row_v, t_v, cvb, cib):
    wid = lax.axis_index("s") * SC_NC + lax.axis_index("c")
    base = wid * qpw
    pltpu.sync_copy(t_hbm.at[pl.ds(base, qpw)], t_v)

    inf16 = jnp.full((SC_L,), jnp.inf, jnp.float32)
    big16 = jnp.full((SC_L,), _BIG_IDX, jnp.int32)
    iota16 = lax.iota(jnp.int32, SC_L)

    def per_query(qi, carry):
        qg = base + qi
        pltpu.sync_copy(d2_hbm.at[qg], row_v)
        tblk = t_v[pl.ds((qi // SC_L) * SC_L, SC_L)]
        tsel = jnp.where(iota16 == qi % SC_L, tblk, -jnp.inf)
        thr = jnp.max(tsel)

        def init_step(n, c):
            cvb[pl.ds(n * SC_L, SC_L)] = inf16
            cib[pl.ds(n * SC_L, SC_L)] = big16
            return c
        lax.fori_loop(0, (CAP + SC_L) // SC_L, init_step, 0)

        def scan_group(g, ptr):
            vs, ms, cnts = [], [], []
            for u in range(SC_G):
                v = row_v[pl.ds((g * SC_G + u) * SC_L, SC_L)]
                m = v <= thr
                vs.append(v)
                ms.append(m)
                cnts.append(jnp.max(plsc.all_reduce_population_count(m)))
            for u in range(SC_G):
                gi = iota16 + (g * SC_G + u) * SC_L
                plsc.store_compressed(cvb.at[pl.ds(ptr, SC_L)], vs[u],
                                      mask=ms[u])
                plsc.store_compressed(cib.at[pl.ds(ptr, SC_L)], gi,
                                      mask=ms[u])
                ptr = jnp.minimum(ptr + cnts[u], CAP)
            return ptr
        lax.fori_loop(0, vpad // (SC_L * SC_G), scan_group, jnp.int32(0))

        pltpu.sync_copy(cvb.at[pl.ds(0, CAP)], cv_hbm.at[qg])
        pltpu.sync_copy(cib.at[pl.ds(0, CAP)], ci_hbm.at[qg])
        return carry

    lax.fori_loop(0, qpw, per_query, 0)


def _finish_body(n_chunks, cv_ref, ci_ref, d_ref, i_ref, bufv, bufi):
    j = pl.program_id(1)

    @pl.when(j == 0)
    def _init():
        bufv[...] = jnp.full(bufv.shape, jnp.inf, jnp.float32)
        bufi[...] = jnp.full(bufi.shape, _BIG_IDX, jnp.int32)

    v = cv_ref[...]
    live = jnp.any(v[:, 0:1] != jnp.inf)

    @pl.when(live)
    def _do_merge():
        idx = ci_ref[...]
        lane = lax.broadcasted_iota(jnp.int32, v.shape, 1)
        cv, ci = _sort_lanes(v, idx, lane, ascending=False)
        nv, ni = _merge_desc_into_asc(bufv[...], bufi[...], cv, ci, lane)
        bufv[...] = nv
        bufi[...] = ni

    @pl.when(j == n_chunks - 1)
    def _fin():
        d_ref[...] = bufv[...]
        i_ref[...] = bufi[...]


NH = 4


def kernel(x, W):
    B, d = x.shape
    V, _ = W.shape
    n_chunks = -(-V // LANES)
    if n_chunks % SS:
        n_chunks += SS - n_chunks % SS
    v_pad = n_chunks * LANES
    xsq = jnp.sum(x * x, axis=1, keepdims=True)
    wsq = jnp.sum(W * W, axis=1)
    if v_pad != V:
        W = jnp.pad(W, ((0, v_pad - V), (0, 0)))
        wsq = jnp.pad(wsq, (0, v_pad - V))
    wsq3 = wsq.reshape(n_chunks, 1, LANES)
    bb = 256
    hb = B // NH

    body = functools.partial(_score_body, n_chunks, V)
    n_workers = SC_NC * SC_NS
    qpw = hb // n_workers
    mesh = plsc.VectorSubcoreMesh(core_axis_name="c", subcore_axis_name="s")
    sc_body = functools.partial(_sc_filter_body, qpw, v_pad)
    fin_chunks = CAP // LANES
    fbody = functools.partial(_finish_body, fin_chunks)

    ds, is_ = [], []
    for h in range(NH):
        xh = x[h * hb:(h + 1) * hb]
        xsqh = xsq[h * hb:(h + 1) * hb]
        d2, t = pl.pallas_call(
            body,
            grid=(hb // bb, n_chunks),
            in_specs=[
                pl.BlockSpec((bb, d), lambda i, j: (i, 0)),
                pl.BlockSpec((LANES, d), lambda i, j: (j, 0)),
                pl.BlockSpec((bb, 1), lambda i, j: (i, 0)),
                pl.BlockSpec((1, 1, LANES), lambda i, j: (j, 0, 0)),
            ],
            out_specs=[
                pl.BlockSpec((bb, LANES), lambda i, j: (i, j)),
                pl.BlockSpec((bb, 1), lambda i, j: (i, 0)),
            ],
            out_shape=[
                jax.ShapeDtypeStruct((hb, v_pad), jnp.float32),
                jax.ShapeDtypeStruct((hb, 1), jnp.float32),
            ],
            scratch_shapes=[
                pltpu.VMEM((bb, LANES), jnp.float32),
            ],
            compiler_params=pltpu.CompilerParams(
                dimension_semantics=("arbitrary", "arbitrary")),
        )(xh, W, xsqh, wsq3)

        cand_v, cand_i = pl.kernel(
            sc_body,
            out_type=[
                jax.ShapeDtypeStruct((hb, CAP), jnp.float32),
                jax.ShapeDtypeStruct((hb, CAP), jnp.int32),
            ],
            mesh=mesh,
            scratch_types=[
                pltpu.VMEM((v_pad,), jnp.float32),
                pltpu.VMEM((qpw,), jnp.float32),
                pltpu.VMEM((CAP + SC_L,), jnp.float32),
                pltpu.VMEM((CAP + SC_L,), jnp.int32),
            ],
            compiler_params=pltpu.CompilerParams(needs_layout_passes=False),
        )(d2, t.reshape(hb))

        D, I = pl.pallas_call(
            fbody,
            grid=(hb // bb, fin_chunks),
            in_specs=[
                pl.BlockSpec((bb, LANES), lambda i, j: (i, j)),
                pl.BlockSpec((bb, LANES), lambda i, j: (i, j)),
            ],
            out_specs=[
                pl.BlockSpec((bb, K_NB), lambda i, j: (i, 0)),
                pl.BlockSpec((bb, K_NB), lambda i, j: (i, 0)),
            ],
            out_shape=[
                jax.ShapeDtypeStruct((hb, K_NB), jnp.float32),
                jax.ShapeDtypeStruct((hb, K_NB), jnp.int32),
            ],
            scratch_shapes=[
                pltpu.VMEM((bb, LANES), jnp.float32),
                pltpu.VMEM((bb, LANES), jnp.int32),
            ],
            compiler_params=pltpu.CompilerParams(
                dimension_semantics=("arbitrary", "arbitrary")),
        )(cand_v, cand_i)
        ds.append(D)
        is_.append(I)

    return x, jnp.concatenate(ds, axis=0), jnp.concatenate(is_, axis=0)
